# Initial kernel scaffold; baseline (speedup 1.0000x reference)
import jax
import jax.numpy as jnp
from jax.experimental import pallas as pl


def kernel(x, edge_index, Wl1, Wr1, b1, Wl2, Wr2, b2):
    raise NotImplementedError("write your pallas kernel here")


# speedup vs baseline: 5.6576x; 5.6576x over previous
import functools

import jax
import jax.numpy as jnp
from jax import lax
from jax.experimental import pallas as pl
from jax.experimental.pallas import tpu as pltpu
from jax.experimental.pallas import tpu_sc as plsc

N_NODES = 10000
N_EDGES = 320000
D = 128

NC = 2
NS = 16
NW = NC * NS
PER_TILE = N_EDGES // NW
CHUNK = 80
N_CHUNKS = PER_TILE // CHUNK
NPAD = 10240
ROWS_PER_TILE = NPAD // NS


def _sc_agg_body(with_cnt, x_hbm, src_hbm, dst_hbm, *rest):
    if with_cnt:
        (p_out, c_out, agg_sh, cnt_sh, src_v, dst_v, rows_v, ones_v,
         z_v, sem) = rest
    else:
        (p_out, agg_sh, src_v, dst_v, rows_v, sem) = rest
        c_out = cnt_sh = ones_v = z_v = None

    cid = lax.axis_index("c")
    sid = lax.axis_index("s")
    wid = sid * NC + cid

    def zrows_body(r, carry):
        for j in range(D // 16):
            rows_v[r, pl.ds(j * 16, 16)] = jnp.zeros((16,), jnp.float32)
        return carry

    lax.fori_loop(0, CHUNK, zrows_body, 0)

    if with_cnt:
        def zcnt_body(k, carry):
            z_v[pl.ds(k * 16, 16)] = jnp.zeros((16,), jnp.float32)
            return carry

        lax.fori_loop(0, ROWS_PER_TILE // 16, zcnt_body, 0)

        def ones_body(k, carry):
            ones_v[pl.ds(k * 16, 16)] = jnp.ones((16,), jnp.float32)
            return carry

        lax.fori_loop(0, CHUNK // 16, ones_body, 0)

    row0 = sid * ROWS_PER_TILE
    for i in range(ROWS_PER_TILE // CHUNK):
        pltpu.sync_copy(rows_v, agg_sh.at[pl.ds(row0 + i * CHUNK, CHUNK)])
    if with_cnt:
        pltpu.sync_copy(z_v, cnt_sh.at[pl.ds(row0, ROWS_PER_TILE)])

    plsc.subcore_barrier()

    ebase = wid * PER_TILE
    ones16 = jnp.ones((16,), jnp.float32)

    def step(g, carry):
        base = ebase + g * CHUNK
        pltpu.sync_copy(src_hbm.at[pl.ds(base, CHUNK)], src_v)
        pltpu.sync_copy(dst_hbm.at[pl.ds(base, CHUNK)], dst_v)
        pltpu.async_copy(x_hbm.at[src_v], rows_v, sem).wait()
        pltpu.sync_copy(rows_v, agg_sh.at[dst_v], add=True)
        if with_cnt:
            pltpu.sync_copy(ones_v, cnt_sh.at[dst_v], add=True)
        return carry

    lax.fori_loop(0, N_CHUNKS, step, 0)

    plsc.subcore_barrier()
    pltpu.sync_copy(agg_sh.at[pl.ds(row0, ROWS_PER_TILE)],
                    p_out.at[cid, pl.ds(row0, ROWS_PER_TILE)])
    if with_cnt:
        pltpu.sync_copy(cnt_sh.at[pl.ds(row0, ROWS_PER_TILE)],
                        c_out.at[cid, 0, pl.ds(row0, ROWS_PER_TILE)])


def _make_sc_agg(with_cnt):
    out_type = [jax.ShapeDtypeStruct((NC, NPAD, D), jnp.float32)]
    if with_cnt:
        out_type.append(jax.ShapeDtypeStruct((NC, 8, NPAD), jnp.float32))
    scratch = [
        pltpu.VMEM_SHARED((NPAD, D), jnp.float32),
    ]
    if with_cnt:
        scratch.append(pltpu.VMEM_SHARED((NPAD,), jnp.float32))
    scratch += [
        pltpu.VMEM((CHUNK,), jnp.int32),
        pltpu.VMEM((CHUNK,), jnp.int32),
        pltpu.VMEM((CHUNK, D), jnp.float32),
    ]
    if with_cnt:
        scratch.append(pltpu.VMEM((CHUNK,), jnp.float32))
        scratch.append(pltpu.VMEM((ROWS_PER_TILE,), jnp.float32))
    scratch.append(pltpu.SemaphoreType.DMA)

    return pl.kernel(
        functools.partial(_sc_agg_body, with_cnt),
        mesh=plsc.VectorSubcoreMesh(core_axis_name="c", subcore_axis_name="s"),
        out_type=out_type,
        scratch_types=scratch,
    )


_SC_AGG_CACHE = {}


def _get_sc_agg(with_cnt):
    if with_cnt not in _SC_AGG_CACHE:
        _SC_AGG_CACHE[with_cnt] = _make_sc_agg(with_cnt)
    return _SC_AGG_CACHE[with_cnt]


TC_BLOCK = 1000
NBLK = N_NODES // TC_BLOCK


def _tc_layer_body(relu, p_ref, c_ref, x_ref, wl_ref, wr_ref, b_ref, o_ref):
    agg = p_ref[0] + p_ref[1]
    cnt = jnp.sum(c_ref[0], axis=0)[:, None]
    inv = 1.0 / jnp.maximum(cnt, 1.0)
    dn = (((1,), (1,)), ((), ()))
    acc = lax.dot_general(agg * inv, wl_ref[...], dn,
                          preferred_element_type=jnp.float32)
    acc += lax.dot_general(x_ref[...], wr_ref[...], dn,
                           preferred_element_type=jnp.float32)
    acc += b_ref[...]
    if relu:
        acc = jnp.maximum(acc, 0.0)
    o_ref[...] = acc


def _tc_layer(p, c3, x, wl, wr, b, relu):
    return pl.pallas_call(
        functools.partial(_tc_layer_body, relu),
        grid=(NBLK,),
        in_specs=[
            pl.BlockSpec((NC, TC_BLOCK, D), lambda i: (0, i, 0)),
            pl.BlockSpec((1, NC, TC_BLOCK), lambda i: (i, 0, 0)),
            pl.BlockSpec((TC_BLOCK, D), lambda i: (i, 0)),
            pl.BlockSpec((D, D), lambda i: (0, 0)),
            pl.BlockSpec((D, D), lambda i: (0, 0)),
            pl.BlockSpec((1, D), lambda i: (0, 0)),
        ],
        out_specs=pl.BlockSpec((TC_BLOCK, D), lambda i: (i, 0)),
        out_shape=jax.ShapeDtypeStruct((N_NODES, D), jnp.float32),
    )(p, c3, x, wl, wr, b.reshape(1, D))


def kernel(x, edge_index, Wl1, Wr1, b1, Wl2, Wr2, b2):
    src = edge_index[0].astype(jnp.int32)
    dst = edge_index[1].astype(jnp.int32)
    p1, craw = _get_sc_agg(True)(x, src, dst)
    cnt = craw[:, 0, :N_NODES]
    c3 = cnt.reshape(NC, NBLK, TC_BLOCK).transpose(1, 0, 2)
    h = _tc_layer(p1, c3, x, Wl1, Wr1, b1, relu=True)
    (p2,) = _get_sc_agg(False)(h, src, dst)
    out = _tc_layer(p2, c3, h, Wl2, Wr2, b2, relu=False)
    return out


# speedup vs baseline: 9.0192x; 1.5942x over previous
import functools

import jax
import jax.numpy as jnp
from jax import lax
from jax.experimental import pallas as pl
from jax.experimental.pallas import tpu as pltpu
from jax.experimental.pallas import tpu_sc as plsc

N_NODES = 10000
N_EDGES = 320000
D = 128

NC = 2
NS = 16
NW = NC * NS
PER_TILE = N_EDGES // NW
CHUNK = 80
N_CHUNKS = PER_TILE // CHUNK
NPAD = 10240
ROWS_PER_TILE = NPAD // NS


def _sc_agg_body(with_cnt, x_hbm, src_hbm, dst_hbm, *rest):
    if with_cnt:
        (p_out, c_out, agg_sh, cnt_sh, src0, src1, dst0, dst1, rows0, rows1,
         ones_v, z_v, sem0, sem1) = rest
    else:
        (p_out, agg_sh, src0, src1, dst0, dst1, rows0, rows1,
         sem0, sem1) = rest
        c_out = cnt_sh = ones_v = z_v = None
    srcs = (src0, src1)
    dsts = (dst0, dst1)
    rows = (rows0, rows1)
    sems = (sem0, sem1)

    cid = lax.axis_index("c")
    sid = lax.axis_index("s")
    wid = sid * NC + cid

    def zrows_body(r, carry):
        for j in range(D // 16):
            rows0[r, pl.ds(j * 16, 16)] = jnp.zeros((16,), jnp.float32)
        return carry

    lax.fori_loop(0, CHUNK, zrows_body, 0)

    if with_cnt:
        def zcnt_body(k, carry):
            z_v[pl.ds(k * 16, 16)] = jnp.zeros((16,), jnp.float32)
            return carry

        lax.fori_loop(0, ROWS_PER_TILE // 16, zcnt_body, 0)

        def ones_body(k, carry):
            ones_v[pl.ds(k * 16, 16)] = jnp.ones((16,), jnp.float32)
            return carry

        lax.fori_loop(0, CHUNK // 16, ones_body, 0)

    row0 = sid * ROWS_PER_TILE
    for i in range(ROWS_PER_TILE // CHUNK):
        pltpu.sync_copy(rows0, agg_sh.at[pl.ds(row0 + i * CHUNK, CHUNK)])
    if with_cnt:
        pltpu.sync_copy(z_v, cnt_sh.at[pl.ds(row0, ROWS_PER_TILE)])

    plsc.subcore_barrier()

    ebase = wid * PER_TILE

    def load_idx(g, b):
        base = ebase + g * CHUNK
        pltpu.sync_copy(src_hbm.at[pl.ds(base, CHUNK)], srcs[b])
        pltpu.sync_copy(dst_hbm.at[pl.ds(base, CHUNK)], dsts[b])

    def start(b):
        pltpu.async_copy(x_hbm.at[srcs[b]], rows[b], sems[b])

    def wait(b):
        pltpu.make_async_copy(x_hbm.at[srcs[b]], rows[b], sems[b]).wait()

    def scat(b):
        pltpu.sync_copy(rows[b], agg_sh.at[dsts[b]], add=True)
        if with_cnt:
            pltpu.sync_copy(ones_v, cnt_sh.at[dsts[b]], add=True)

    load_idx(0, 0)
    start(0)

    def body(i, carry):
        g0 = 2 * i
        load_idx(g0 + 1, 1)
        start(1)
        wait(0)
        scat(0)
        load_idx(g0 + 2, 0)
        start(0)
        wait(1)
        scat(1)
        return carry

    lax.fori_loop(0, (N_CHUNKS - 1) // 2, body, 0)
    wait(0)
    scat(0)

    plsc.subcore_barrier()
    pltpu.sync_copy(agg_sh.at[pl.ds(row0, ROWS_PER_TILE)],
                    p_out.at[cid, pl.ds(row0, ROWS_PER_TILE)])
    if with_cnt:
        pltpu.sync_copy(cnt_sh.at[pl.ds(row0, ROWS_PER_TILE)],
                        c_out.at[cid, 0, pl.ds(row0, ROWS_PER_TILE)])


def _make_sc_agg(with_cnt):
    out_type = [jax.ShapeDtypeStruct((NC, NPAD, D), jnp.float32)]
    if with_cnt:
        out_type.append(jax.ShapeDtypeStruct((NC, 8, NPAD), jnp.float32))
    scratch = [
        pltpu.VMEM_SHARED((NPAD, D), jnp.float32),
    ]
    if with_cnt:
        scratch.append(pltpu.VMEM_SHARED((NPAD,), jnp.float32))
    scratch += [
        pltpu.VMEM((CHUNK,), jnp.int32),
        pltpu.VMEM((CHUNK,), jnp.int32),
        pltpu.VMEM((CHUNK,), jnp.int32),
        pltpu.VMEM((CHUNK,), jnp.int32),
        pltpu.VMEM((CHUNK, D), jnp.float32),
        pltpu.VMEM((CHUNK, D), jnp.float32),
    ]
    if with_cnt:
        scratch.append(pltpu.VMEM((CHUNK,), jnp.float32))
        scratch.append(pltpu.VMEM((ROWS_PER_TILE,), jnp.float32))
    scratch.append(pltpu.SemaphoreType.DMA)
    scratch.append(pltpu.SemaphoreType.DMA)

    return pl.kernel(
        functools.partial(_sc_agg_body, with_cnt),
        mesh=plsc.VectorSubcoreMesh(core_axis_name="c", subcore_axis_name="s"),
        out_type=out_type,
        scratch_types=scratch,
    )


_SC_AGG_CACHE = {}


def _get_sc_agg(with_cnt):
    if with_cnt not in _SC_AGG_CACHE:
        _SC_AGG_CACHE[with_cnt] = _make_sc_agg(with_cnt)
    return _SC_AGG_CACHE[with_cnt]


TC_BLOCK = 1000
NBLK = N_NODES // TC_BLOCK


def _tc_layer_body(relu, p_ref, c_ref, x_ref, wl_ref, wr_ref, b_ref, o_ref):
    agg = p_ref[0] + p_ref[1]
    cnt = jnp.sum(c_ref[0], axis=0)[:, None]
    inv = 1.0 / jnp.maximum(cnt, 1.0)
    dn = (((1,), (1,)), ((), ()))
    acc = lax.dot_general(agg * inv, wl_ref[...], dn,
                          preferred_element_type=jnp.float32)
    acc += lax.dot_general(x_ref[...], wr_ref[...], dn,
                           preferred_element_type=jnp.float32)
    acc += b_ref[...]
    if relu:
        acc = jnp.maximum(acc, 0.0)
    o_ref[...] = acc


def _tc_layer(p, c3, x, wl, wr, b, relu):
    return pl.pallas_call(
        functools.partial(_tc_layer_body, relu),
        grid=(NBLK,),
        in_specs=[
            pl.BlockSpec((NC, TC_BLOCK, D), lambda i: (0, i, 0)),
            pl.BlockSpec((1, NC, TC_BLOCK), lambda i: (i, 0, 0)),
            pl.BlockSpec((TC_BLOCK, D), lambda i: (i, 0)),
            pl.BlockSpec((D, D), lambda i: (0, 0)),
            pl.BlockSpec((D, D), lambda i: (0, 0)),
            pl.BlockSpec((1, D), lambda i: (0, 0)),
        ],
        out_specs=pl.BlockSpec((TC_BLOCK, D), lambda i: (i, 0)),
        out_shape=jax.ShapeDtypeStruct((N_NODES, D), jnp.float32),
    )(p, c3, x, wl, wr, b.reshape(1, D))


def kernel(x, edge_index, Wl1, Wr1, b1, Wl2, Wr2, b2):
    src = edge_index[0].astype(jnp.int32)
    dst = edge_index[1].astype(jnp.int32)
    p1, craw = _get_sc_agg(True)(x, src, dst)
    cnt = craw[:, 0, :N_NODES]
    c3 = cnt.reshape(NC, NBLK, TC_BLOCK).transpose(1, 0, 2)
    h = _tc_layer(p1, c3, x, Wl1, Wr1, b1, relu=True)
    (p2,) = _get_sc_agg(False)(h, src, dst)
    out = _tc_layer(p2, c3, h, Wl2, Wr2, b2, relu=False)
    return out


# speedup vs baseline: 10.9946x; 1.2190x over previous
import functools

import jax
import jax.numpy as jnp
from jax import lax
from jax.experimental import pallas as pl
from jax.experimental.pallas import tpu as pltpu
from jax.experimental.pallas import tpu_sc as plsc

N_NODES = 10000
N_EDGES = 320000
D = 128

NC = 2
NS = 16
NW = NC * NS
PER_TILE = N_EDGES // NW
CHUNK = 80
N_CHUNKS = PER_TILE // CHUNK
NPAD = 10240
ROWS_PER_TILE = NPAD // NS


def _sc_agg_body(with_cnt, x_hbm, src_hbm, dst_hbm, *rest):
    if with_cnt:
        (p_out, c_out, agg_sh, cnt_sh, src0, src1, dst0, dst1, rows0, rows1,
         ones_v, z_v, sem0, sem1, isem0, isem1) = rest
    else:
        (p_out, agg_sh, src0, src1, dst0, dst1, rows0, rows1,
         sem0, sem1, isem0, isem1) = rest
        c_out = cnt_sh = ones_v = z_v = None
    srcs = (src0, src1)
    dsts = (dst0, dst1)
    rows = (rows0, rows1)
    sems = (sem0, sem1)
    isems = (isem0, isem1)

    cid = lax.axis_index("c")
    sid = lax.axis_index("s")
    wid = sid * NC + cid

    def zrows_body(r, carry):
        for j in range(D // 16):
            rows0[r, pl.ds(j * 16, 16)] = jnp.zeros((16,), jnp.float32)
        return carry

    lax.fori_loop(0, CHUNK, zrows_body, 0)

    if with_cnt:
        def zcnt_body(k, carry):
            z_v[pl.ds(k * 16, 16)] = jnp.zeros((16,), jnp.float32)
            return carry

        lax.fori_loop(0, ROWS_PER_TILE // 16, zcnt_body, 0)

        def ones_body(k, carry):
            ones_v[pl.ds(k * 16, 16)] = jnp.ones((16,), jnp.float32)
            return carry

        lax.fori_loop(0, CHUNK // 16, ones_body, 0)

    row0 = sid * ROWS_PER_TILE
    for i in range(ROWS_PER_TILE // CHUNK):
        pltpu.sync_copy(rows0, agg_sh.at[pl.ds(row0 + i * CHUNK, CHUNK)])
    if with_cnt:
        pltpu.sync_copy(z_v, cnt_sh.at[pl.ds(row0, ROWS_PER_TILE)])

    plsc.subcore_barrier()

    ebase = wid * PER_TILE

    def start_src(g, b):
        base = ebase + g * CHUNK
        pltpu.async_copy(src_hbm.at[pl.ds(base, CHUNK)], srcs[b], isems[b])

    def start_dst(g, b):
        base = ebase + g * CHUNK
        pltpu.async_copy(dst_hbm.at[pl.ds(base, CHUNK)], dsts[b], isems[b])

    def wait_idx(g, b):
        base = ebase + g * CHUNK
        pltpu.make_async_copy(src_hbm.at[pl.ds(base, CHUNK)], srcs[b],
                              isems[b]).wait()
        pltpu.make_async_copy(dst_hbm.at[pl.ds(base, CHUNK)], dsts[b],
                              isems[b]).wait()

    def start(b):
        pltpu.async_copy(x_hbm.at[srcs[b]], rows[b], sems[b])

    def wait(b):
        pltpu.make_async_copy(x_hbm.at[srcs[b]], rows[b], sems[b]).wait()

    def scat(b):
        pltpu.sync_copy(rows[b], agg_sh.at[dsts[b]], add=True)
        if with_cnt:
            pltpu.sync_copy(ones_v, cnt_sh.at[dsts[b]], add=True)

    start_src(0, 0)
    start_dst(0, 0)
    wait_idx(0, 0)
    start(0)
    start_src(1, 1)
    start_dst(1, 1)

    def body(i, carry):
        g0 = 2 * i
        wait_idx(g0 + 1, 1)
        start(1)
        wait(0)
        start_src(g0 + 2, 0)
        scat(0)
        start_dst(g0 + 2, 0)
        wait(1)
        wait_idx(g0 + 2, 0)
        start(0)
        scat(1)

        @pl.when(g0 + 3 < N_CHUNKS)
        def _():
            start_src(g0 + 3, 1)
            start_dst(g0 + 3, 1)

        return carry

    lax.fori_loop(0, (N_CHUNKS - 1) // 2, body, 0)
    wait(0)
    scat(0)

    plsc.subcore_barrier()
    pltpu.sync_copy(agg_sh.at[pl.ds(row0, ROWS_PER_TILE)],
                    p_out.at[cid, pl.ds(row0, ROWS_PER_TILE)])
    if with_cnt:
        pltpu.sync_copy(cnt_sh.at[pl.ds(row0, ROWS_PER_TILE)],
                        c_out.at[cid, 0, pl.ds(row0, ROWS_PER_TILE)])


def _make_sc_agg(with_cnt):
    out_type = [jax.ShapeDtypeStruct((NC, NPAD, D), jnp.float32)]
    if with_cnt:
        out_type.append(jax.ShapeDtypeStruct((NC, 8, NPAD), jnp.float32))
    scratch = [
        pltpu.VMEM_SHARED((NPAD, D), jnp.float32),
    ]
    if with_cnt:
        scratch.append(pltpu.VMEM_SHARED((NPAD,), jnp.float32))
    scratch += [
        pltpu.VMEM((CHUNK,), jnp.int32),
        pltpu.VMEM((CHUNK,), jnp.int32),
        pltpu.VMEM((CHUNK,), jnp.int32),
        pltpu.VMEM((CHUNK,), jnp.int32),
        pltpu.VMEM((CHUNK, D), jnp.float32),
        pltpu.VMEM((CHUNK, D), jnp.float32),
    ]
    if with_cnt:
        scratch.append(pltpu.VMEM((CHUNK,), jnp.float32))
        scratch.append(pltpu.VMEM((ROWS_PER_TILE,), jnp.float32))
    scratch.append(pltpu.SemaphoreType.DMA)
    scratch.append(pltpu.SemaphoreType.DMA)
    scratch.append(pltpu.SemaphoreType.DMA)
    scratch.append(pltpu.SemaphoreType.DMA)

    return pl.kernel(
        functools.partial(_sc_agg_body, with_cnt),
        mesh=plsc.VectorSubcoreMesh(core_axis_name="c", subcore_axis_name="s"),
        out_type=out_type,
        scratch_types=scratch,
    )


_SC_AGG_CACHE = {}


def _get_sc_agg(with_cnt):
    if with_cnt not in _SC_AGG_CACHE:
        _SC_AGG_CACHE[with_cnt] = _make_sc_agg(with_cnt)
    return _SC_AGG_CACHE[with_cnt]


TC_BLOCK = 1000
NBLK = N_NODES // TC_BLOCK


def _tc_layer_body(relu, p_ref, c_ref, x_ref, wl_ref, wr_ref, b_ref, o_ref):
    agg = p_ref[0] + p_ref[1]
    cnt = jnp.sum(c_ref[0], axis=0)[:, None]
    inv = 1.0 / jnp.maximum(cnt, 1.0)
    dn = (((1,), (1,)), ((), ()))
    acc = lax.dot_general(agg * inv, wl_ref[...], dn,
                          preferred_element_type=jnp.float32)
    acc += lax.dot_general(x_ref[...], wr_ref[...], dn,
                           preferred_element_type=jnp.float32)
    acc += b_ref[...]
    if relu:
        acc = jnp.maximum(acc, 0.0)
    o_ref[...] = acc


def _tc_layer(p, c3, x, wl, wr, b, relu):
    return pl.pallas_call(
        functools.partial(_tc_layer_body, relu),
        grid=(NBLK,),
        in_specs=[
            pl.BlockSpec((NC, TC_BLOCK, D), lambda i: (0, i, 0)),
            pl.BlockSpec((1, NC, TC_BLOCK), lambda i: (i, 0, 0)),
            pl.BlockSpec((TC_BLOCK, D), lambda i: (i, 0)),
            pl.BlockSpec((D, D), lambda i: (0, 0)),
            pl.BlockSpec((D, D), lambda i: (0, 0)),
            pl.BlockSpec((1, D), lambda i: (0, 0)),
        ],
        out_specs=pl.BlockSpec((TC_BLOCK, D), lambda i: (i, 0)),
        out_shape=jax.ShapeDtypeStruct((N_NODES, D), jnp.float32),
    )(p, c3, x, wl, wr, b.reshape(1, D))


def kernel(x, edge_index, Wl1, Wr1, b1, Wl2, Wr2, b2):
    src = edge_index[0].astype(jnp.int32)
    dst = edge_index[1].astype(jnp.int32)
    p1, craw = _get_sc_agg(True)(x, src, dst)
    cnt = craw[:, 0, :N_NODES]
    c3 = cnt.reshape(NC, NBLK, TC_BLOCK).transpose(1, 0, 2)
    h = _tc_layer(p1, c3, x, Wl1, Wr1, b1, relu=True)
    (p2,) = _get_sc_agg(False)(h, src, dst)
    out = _tc_layer(p2, c3, h, Wl2, Wr2, b2, relu=False)
    return out


# speedup vs baseline: 12.8798x; 1.1715x over previous
import functools

import jax
import jax.numpy as jnp
from jax import lax
from jax.experimental import pallas as pl
from jax.experimental.pallas import tpu as pltpu
from jax.experimental.pallas import tpu_sc as plsc

N_NODES = 10000
N_EDGES = 320000
D = 128

NC = 2
NS = 16
NW = NC * NS
PER_TILE = N_EDGES // NW
CHUNK = 80
N_CHUNKS = PER_TILE // CHUNK
NPAD = 10240
ROWS_PER_TILE = NPAD // NS


def _sc_agg_body(with_cnt, x_hbm, src_hbm, dst_hbm, *rest):
    if with_cnt:
        (p_out, c_out, agg_sh, cnt_sh, src0, src1, dst0, dst1, dst2, dst3,
         rows0, rows1, ones_v, z_v, gsem0, gsem1, isS0, isS1,
         isD0, isD1, isD2, isD3, ss0, ss1, ss2, ss3) = rest
    else:
        (p_out, agg_sh, src0, src1, dst0, dst1, dst2, dst3,
         rows0, rows1, gsem0, gsem1, isS0, isS1,
         isD0, isD1, isD2, isD3, ss0, ss1, ss2, ss3) = rest
        c_out = cnt_sh = ones_v = z_v = None
    srcs = (src0, src1)
    dsts = (dst0, dst1, dst2, dst3)
    rows = (rows0, rows1)
    gsems = (gsem0, gsem1)
    isems_s = (isS0, isS1)
    isems_d = (isD0, isD1, isD2, isD3)
    ssems = (ss0, ss1, ss2, ss3)

    cid = lax.axis_index("c")
    sid = lax.axis_index("s")
    wid = sid * NC + cid

    def zrows_body(r, carry):
        for j in range(D // 16):
            rows0[r, pl.ds(j * 16, 16)] = jnp.zeros((16,), jnp.float32)
        return carry

    lax.fori_loop(0, CHUNK, zrows_body, 0)

    if with_cnt:
        def zcnt_body(k, carry):
            z_v[pl.ds(k * 16, 16)] = jnp.zeros((16,), jnp.float32)
            return carry

        lax.fori_loop(0, ROWS_PER_TILE // 16, zcnt_body, 0)

        def ones_body(k, carry):
            ones_v[pl.ds(k * 16, 16)] = jnp.ones((16,), jnp.float32)
            return carry

        lax.fori_loop(0, CHUNK // 16, ones_body, 0)

    row0 = sid * ROWS_PER_TILE
    for i in range(ROWS_PER_TILE // CHUNK):
        pltpu.sync_copy(rows0, agg_sh.at[pl.ds(row0 + i * CHUNK, CHUNK)])
    if with_cnt:
        pltpu.sync_copy(z_v, cnt_sh.at[pl.ds(row0, ROWS_PER_TILE)])

    plsc.subcore_barrier()

    ebase = wid * PER_TILE

    def start_src(g, p):
        pltpu.async_copy(src_hbm.at[pl.ds(ebase + g * CHUNK, CHUNK)],
                         srcs[p], isems_s[p])

    def start_dst(g, q):
        pltpu.async_copy(dst_hbm.at[pl.ds(ebase + g * CHUNK, CHUNK)],
                         dsts[q], isems_d[q])

    def wait_src(g, p):
        pltpu.make_async_copy(src_hbm.at[pl.ds(ebase + g * CHUNK, CHUNK)],
                              srcs[p], isems_s[p]).wait()

    def wait_dst(g, q):
        pltpu.make_async_copy(dst_hbm.at[pl.ds(ebase + g * CHUNK, CHUNK)],
                              dsts[q], isems_d[q]).wait()

    def start_gather(p):
        pltpu.async_copy(x_hbm.at[srcs[p]], rows[p], gsems[p])

    def wait_gather(p):
        pltpu.make_async_copy(x_hbm.at[srcs[p]], rows[p], gsems[p]).wait()

    def start_scat(q):
        pltpu.async_copy(rows[q % 2], agg_sh.at[dsts[q]], ssems[q], add=True)
        if with_cnt:
            pltpu.async_copy(ones_v, cnt_sh.at[dsts[q]], ssems[q], add=True)

    def wait_scat(q):
        pltpu.make_async_copy(rows[q % 2], agg_sh.at[dsts[q]], ssems[q]).wait()
        if with_cnt:
            pltpu.make_async_copy(ones_v, cnt_sh.at[dsts[q]], ssems[q]).wait()

    def ops(g, k, first=False, n_left=3):
        p, q = k % 2, k
        if not first:
            wait_scat((q + 3) % 4)
        if n_left >= 1:
            wait_src(g + 1, (p + 1) % 2)
            wait_dst(g + 1, (q + 1) % 4)
            start_gather((p + 1) % 2)
        wait_gather(p)
        if n_left >= 2:
            start_src(g + 2, p)
        start_scat(q)
        if n_left >= 2:
            start_dst(g + 2, (q + 2) % 4)

    start_src(0, 0)
    start_dst(0, 0)
    start_src(1, 1)
    start_dst(1, 1)
    wait_src(0, 0)
    wait_dst(0, 0)
    start_gather(0)
    ops(0, 0, first=True)

    def body(i, carry):
        g0 = 4 * i + 1
        for k in range(4):
            ops(g0 + k, (1 + k) % 4)
        return carry

    lax.fori_loop(0, (N_CHUNKS - 5) // 4, body, 0)
    ops(N_CHUNKS - 4, 1)
    ops(N_CHUNKS - 3, 2)
    ops(N_CHUNKS - 2, 3, n_left=1)
    ops(N_CHUNKS - 1, 0, n_left=0)
    wait_scat(0)

    plsc.subcore_barrier()
    pltpu.sync_copy(agg_sh.at[pl.ds(row0, ROWS_PER_TILE)],
                    p_out.at[cid, pl.ds(row0, ROWS_PER_TILE)])
    if with_cnt:
        pltpu.sync_copy(cnt_sh.at[pl.ds(row0, ROWS_PER_TILE)],
                        c_out.at[cid, 0, pl.ds(row0, ROWS_PER_TILE)])


def _make_sc_agg(with_cnt):
    out_type = [jax.ShapeDtypeStruct((NC, NPAD, D), jnp.float32)]
    if with_cnt:
        out_type.append(jax.ShapeDtypeStruct((NC, 8, NPAD), jnp.float32))
    scratch = [
        pltpu.VMEM_SHARED((NPAD, D), jnp.float32),
    ]
    if with_cnt:
        scratch.append(pltpu.VMEM_SHARED((NPAD,), jnp.float32))
    scratch += [
        pltpu.VMEM((CHUNK,), jnp.int32),
        pltpu.VMEM((CHUNK,), jnp.int32),
        pltpu.VMEM((CHUNK,), jnp.int32),
        pltpu.VMEM((CHUNK,), jnp.int32),
        pltpu.VMEM((CHUNK,), jnp.int32),
        pltpu.VMEM((CHUNK,), jnp.int32),
        pltpu.VMEM((CHUNK, D), jnp.float32),
        pltpu.VMEM((CHUNK, D), jnp.float32),
    ]
    if with_cnt:
        scratch.append(pltpu.VMEM((CHUNK,), jnp.float32))
        scratch.append(pltpu.VMEM((ROWS_PER_TILE,), jnp.float32))
    for _ in range(12):
        scratch.append(pltpu.SemaphoreType.DMA)

    return pl.kernel(
        functools.partial(_sc_agg_body, with_cnt),
        mesh=plsc.VectorSubcoreMesh(core_axis_name="c", subcore_axis_name="s"),
        out_type=out_type,
        scratch_types=scratch,
    )


_SC_AGG_CACHE = {}


def _get_sc_agg(with_cnt):
    if with_cnt not in _SC_AGG_CACHE:
        _SC_AGG_CACHE[with_cnt] = _make_sc_agg(with_cnt)
    return _SC_AGG_CACHE[with_cnt]


TC_BLOCK = 1000
NBLK = N_NODES // TC_BLOCK


def _tc_layer_body(relu, p_ref, c_ref, x_ref, wl_ref, wr_ref, b_ref, o_ref):
    agg = p_ref[0] + p_ref[1]
    cnt = jnp.sum(c_ref[0], axis=0)[:, None]
    inv = 1.0 / jnp.maximum(cnt, 1.0)
    dn = (((1,), (1,)), ((), ()))
    acc = lax.dot_general(agg * inv, wl_ref[...], dn,
                          preferred_element_type=jnp.float32)
    acc += lax.dot_general(x_ref[...], wr_ref[...], dn,
                           preferred_element_type=jnp.float32)
    acc += b_ref[...]
    if relu:
        acc = jnp.maximum(acc, 0.0)
    o_ref[...] = acc


def _tc_layer(p, c3, x, wl, wr, b, relu):
    return pl.pallas_call(
        functools.partial(_tc_layer_body, relu),
        grid=(NBLK,),
        in_specs=[
            pl.BlockSpec((NC, TC_BLOCK, D), lambda i: (0, i, 0)),
            pl.BlockSpec((1, NC, TC_BLOCK), lambda i: (i, 0, 0)),
            pl.BlockSpec((TC_BLOCK, D), lambda i: (i, 0)),
            pl.BlockSpec((D, D), lambda i: (0, 0)),
            pl.BlockSpec((D, D), lambda i: (0, 0)),
            pl.BlockSpec((1, D), lambda i: (0, 0)),
        ],
        out_specs=pl.BlockSpec((TC_BLOCK, D), lambda i: (i, 0)),
        out_shape=jax.ShapeDtypeStruct((N_NODES, D), jnp.float32),
    )(p, c3, x, wl, wr, b.reshape(1, D))


def kernel(x, edge_index, Wl1, Wr1, b1, Wl2, Wr2, b2):
    src = edge_index[0].astype(jnp.int32)
    dst = edge_index[1].astype(jnp.int32)
    p1, craw = _get_sc_agg(True)(x, src, dst)
    cnt = craw[:, 0, :N_NODES]
    c3 = cnt.reshape(NC, NBLK, TC_BLOCK).transpose(1, 0, 2)
    h = _tc_layer(p1, c3, x, Wl1, Wr1, b1, relu=True)
    (p2,) = _get_sc_agg(False)(h, src, dst)
    out = _tc_layer(p2, c3, h, Wl2, Wr2, b2, relu=False)
    return out


# speedup vs baseline: 14.3447x; 1.1137x over previous
import functools

import jax
import jax.numpy as jnp
from jax import lax
from jax.experimental import pallas as pl
from jax.experimental.pallas import tpu as pltpu
from jax.experimental.pallas import tpu_sc as plsc

N_NODES = 10000
N_EDGES = 320000
D = 128

NC = 2
NS = 16
NW = NC * NS
PER_TILE = N_EDGES // NW
CHUNK = 128
N_FULL = PER_TILE // CHUNK
TAIL = PER_TILE - N_FULL * CHUNK
NPAD = 10240
ROWS_PER_TILE = NPAD // NS


def _sc_agg_body(with_cnt, x_hbm, src_hbm, dst_hbm, *rest):
    if with_cnt:
        (p_out, c_out, agg_sh, cnt_sh, src0, src1, dst0, dst1, dst2, dst3,
         rows0, rows1, src_t, dst_t, rows_t, ones_v, ones_t, z_v,
         gsem0, gsem1, isS0, isS1, isD0, isD1, isD2, isD3,
         ss0, ss1, ss2, ss3, tsem_i, tsem_g) = rest
    else:
        (p_out, agg_sh, src0, src1, dst0, dst1, dst2, dst3,
         rows0, rows1, src_t, dst_t, rows_t,
         gsem0, gsem1, isS0, isS1, isD0, isD1, isD2, isD3,
         ss0, ss1, ss2, ss3, tsem_i, tsem_g) = rest
        c_out = cnt_sh = ones_v = ones_t = z_v = None
    srcs = (src0, src1)
    dsts = (dst0, dst1, dst2, dst3)
    rows = (rows0, rows1)
    gsems = (gsem0, gsem1)
    isems_s = (isS0, isS1)
    isems_d = (isD0, isD1, isD2, isD3)
    ssems = (ss0, ss1, ss2, ss3)

    cid = lax.axis_index("c")
    sid = lax.axis_index("s")
    wid = sid * NC + cid
    ebase = wid * PER_TILE
    tbase = ebase + N_FULL * CHUNK

    def start_src(g, p):
        pltpu.async_copy(src_hbm.at[pl.ds(ebase + g * CHUNK, CHUNK)],
                         srcs[p], isems_s[p])

    def start_dst(g, q):
        pltpu.async_copy(dst_hbm.at[pl.ds(ebase + g * CHUNK, CHUNK)],
                         dsts[q], isems_d[q])

    def wait_src(g, p):
        pltpu.make_async_copy(src_hbm.at[pl.ds(ebase + g * CHUNK, CHUNK)],
                              srcs[p], isems_s[p]).wait()

    def wait_dst(g, q):
        pltpu.make_async_copy(dst_hbm.at[pl.ds(ebase + g * CHUNK, CHUNK)],
                              dsts[q], isems_d[q]).wait()

    def start_gather(p):
        pltpu.async_copy(x_hbm.at[srcs[p]], rows[p], gsems[p])

    def wait_gather(p):
        pltpu.make_async_copy(x_hbm.at[srcs[p]], rows[p], gsems[p]).wait()

    def start_scat(q):
        pltpu.async_copy(rows[q % 2], agg_sh.at[dsts[q]], ssems[q], add=True)
        if with_cnt:
            pltpu.async_copy(ones_v, cnt_sh.at[dsts[q]], ssems[q], add=True)

    def wait_scat(q):
        pltpu.make_async_copy(rows[q % 2], agg_sh.at[dsts[q]], ssems[q]).wait()
        if with_cnt:
            pltpu.make_async_copy(ones_v, cnt_sh.at[dsts[q]], ssems[q]).wait()

    pltpu.async_copy(src_hbm.at[pl.ds(tbase, TAIL)], src_t, tsem_i)
    pltpu.async_copy(dst_hbm.at[pl.ds(tbase, TAIL)], dst_t, tsem_i)
    start_src(0, 0)
    start_dst(0, 0)
    start_src(1, 1)
    start_dst(1, 1)

    def zrows_body(r, carry):
        for j in range(D // 16):
            rows0[r, pl.ds(j * 16, 16)] = jnp.zeros((16,), jnp.float32)
        return carry

    lax.fori_loop(0, CHUNK, zrows_body, 0)

    if with_cnt:
        def zcnt_body(k, carry):
            z_v[pl.ds(k * 16, 16)] = jnp.zeros((16,), jnp.float32)
            return carry

        lax.fori_loop(0, ROWS_PER_TILE // 16, zcnt_body, 0)

        def ones_body(k, carry):
            ones_v[pl.ds(k * 16, 16)] = jnp.ones((16,), jnp.float32)
            return carry

        lax.fori_loop(0, CHUNK // 16, ones_body, 0)
        ones_t[pl.ds(0, 16)] = jnp.ones((16,), jnp.float32)

    row0 = sid * ROWS_PER_TILE
    for i in range(ROWS_PER_TILE // CHUNK):
        pltpu.sync_copy(rows0, agg_sh.at[pl.ds(row0 + i * CHUNK, CHUNK)])
    if with_cnt:
        pltpu.sync_copy(z_v, cnt_sh.at[pl.ds(row0, ROWS_PER_TILE)])

    plsc.subcore_barrier()

    def ops(g, k, first=False, n_left=3):
        p, q = k % 2, k
        if not first:
            wait_scat((q + 3) % 4)
        if n_left >= 1:
            wait_src(g + 1, (p + 1) % 2)
            wait_dst(g + 1, (q + 1) % 4)
            start_gather((p + 1) % 2)
        wait_gather(p)
        if n_left >= 2:
            start_src(g + 2, p)
        start_scat(q)
        if n_left >= 2:
            start_dst(g + 2, (q + 2) % 4)

    wait_src(0, 0)
    wait_dst(0, 0)
    start_gather(0)
    ops(0, 0, first=True)

    ITERS = (N_FULL - 5) // 4

    def body(i, carry):
        g0 = 4 * i + 1
        for k in range(4):
            ops(g0 + k, (1 + k) % 4)
        return carry

    lax.fori_loop(0, ITERS, body, 0)
    for g in range(4 * ITERS + 1, N_FULL):
        ops(g, g % 4, n_left=min(N_FULL - 1 - g, 3))
    wait_scat((N_FULL - 1) % 4)

    pltpu.make_async_copy(src_hbm.at[pl.ds(tbase, TAIL)], src_t, tsem_i).wait()
    pltpu.make_async_copy(dst_hbm.at[pl.ds(tbase, TAIL)], dst_t, tsem_i).wait()
    pltpu.async_copy(x_hbm.at[src_t], rows_t, tsem_g).wait()
    pltpu.sync_copy(rows_t, agg_sh.at[dst_t], add=True)
    if with_cnt:
        pltpu.sync_copy(ones_t, cnt_sh.at[dst_t], add=True)

    plsc.subcore_barrier()
    pltpu.sync_copy(agg_sh.at[pl.ds(row0, ROWS_PER_TILE)],
                    p_out.at[cid, pl.ds(row0, ROWS_PER_TILE)])
    if with_cnt:
        pltpu.sync_copy(cnt_sh.at[pl.ds(row0, ROWS_PER_TILE)],
                        c_out.at[cid, 0, pl.ds(row0, ROWS_PER_TILE)])


def _make_sc_agg(with_cnt):
    out_type = [jax.ShapeDtypeStruct((NC, NPAD, D), jnp.float32)]
    if with_cnt:
        out_type.append(jax.ShapeDtypeStruct((NC, 8, NPAD), jnp.float32))
    scratch = [
        pltpu.VMEM_SHARED((NPAD, D), jnp.float32),
    ]
    if with_cnt:
        scratch.append(pltpu.VMEM_SHARED((NPAD,), jnp.float32))
    scratch += [
        pltpu.VMEM((CHUNK,), jnp.int32),
        pltpu.VMEM((CHUNK,), jnp.int32),
        pltpu.VMEM((CHUNK,), jnp.int32),
        pltpu.VMEM((CHUNK,), jnp.int32),
        pltpu.VMEM((CHUNK,), jnp.int32),
        pltpu.VMEM((CHUNK,), jnp.int32),
        pltpu.VMEM((CHUNK, D), jnp.float32),
        pltpu.VMEM((CHUNK, D), jnp.float32),
        pltpu.VMEM((TAIL,), jnp.int32),
        pltpu.VMEM((TAIL,), jnp.int32),
        pltpu.VMEM((TAIL, D), jnp.float32),
    ]
    if with_cnt:
        scratch.append(pltpu.VMEM((CHUNK,), jnp.float32))
        scratch.append(pltpu.VMEM((TAIL,), jnp.float32))
        scratch.append(pltpu.VMEM((ROWS_PER_TILE,), jnp.float32))
    for _ in range(14):
        scratch.append(pltpu.SemaphoreType.DMA)

    return pl.kernel(
        functools.partial(_sc_agg_body, with_cnt),
        mesh=plsc.VectorSubcoreMesh(core_axis_name="c", subcore_axis_name="s"),
        out_type=out_type,
        scratch_types=scratch,
    )


_SC_AGG_CACHE = {}


def _get_sc_agg(with_cnt):
    if with_cnt not in _SC_AGG_CACHE:
        _SC_AGG_CACHE[with_cnt] = _make_sc_agg(with_cnt)
    return _SC_AGG_CACHE[with_cnt]


TC_BLOCK = 1000
NBLK = N_NODES // TC_BLOCK


def _tc_layer_body(relu, p_ref, c_ref, x_ref, wl_ref, wr_ref, b_ref, o_ref):
    agg = p_ref[0] + p_ref[1]
    cnt = jnp.sum(c_ref[0], axis=0)[:, None]
    inv = 1.0 / jnp.maximum(cnt, 1.0)
    dn = (((1,), (1,)), ((), ()))
    acc = lax.dot_general(agg * inv, wl_ref[...], dn,
                          preferred_element_type=jnp.float32)
    acc += lax.dot_general(x_ref[...], wr_ref[...], dn,
                           preferred_element_type=jnp.float32)
    acc += b_ref[...]
    if relu:
        acc = jnp.maximum(acc, 0.0)
    o_ref[...] = acc


def _tc_layer(p, c3, x, wl, wr, b, relu):
    return pl.pallas_call(
        functools.partial(_tc_layer_body, relu),
        grid=(NBLK,),
        in_specs=[
            pl.BlockSpec((NC, TC_BLOCK, D), lambda i: (0, i, 0)),
            pl.BlockSpec((1, NC, TC_BLOCK), lambda i: (i, 0, 0)),
            pl.BlockSpec((TC_BLOCK, D), lambda i: (i, 0)),
            pl.BlockSpec((D, D), lambda i: (0, 0)),
            pl.BlockSpec((D, D), lambda i: (0, 0)),
            pl.BlockSpec((1, D), lambda i: (0, 0)),
        ],
        out_specs=pl.BlockSpec((TC_BLOCK, D), lambda i: (i, 0)),
        out_shape=jax.ShapeDtypeStruct((N_NODES, D), jnp.float32),
    )(p, c3, x, wl, wr, b.reshape(1, D))


def kernel(x, edge_index, Wl1, Wr1, b1, Wl2, Wr2, b2):
    src = edge_index[0].astype(jnp.int32)
    dst = edge_index[1].astype(jnp.int32)
    p1, craw = _get_sc_agg(True)(x, src, dst)
    cnt = craw[:, 0, :N_NODES]
    c3 = cnt.reshape(NC, NBLK, TC_BLOCK).transpose(1, 0, 2)
    h = _tc_layer(p1, c3, x, Wl1, Wr1, b1, relu=True)
    (p2,) = _get_sc_agg(False)(h, src, dst)
    out = _tc_layer(p2, c3, h, Wl2, Wr2, b2, relu=False)
    return out


# speedup vs baseline: 14.4819x; 1.0096x over previous
import functools

import jax
import jax.numpy as jnp
from jax import lax
from jax.experimental import pallas as pl
from jax.experimental.pallas import tpu as pltpu
from jax.experimental.pallas import tpu_sc as plsc

N_NODES = 10000
N_EDGES = 320000
D = 128

NC = 2
NS = 16
NW = NC * NS
PER_TILE = N_EDGES // NW
CHUNK = 128
N_FULL = PER_TILE // CHUNK
TAIL = PER_TILE - N_FULL * CHUNK
NPAD = 10240
ROWS_PER_TILE = NPAD // NS


def _sc_agg_body(with_cnt, x_hbm, src_hbm, dst_hbm, *rest):
    if with_cnt:
        (p_out, c_out, agg_sh, cnt_sh, src0, src1, dst0, dst1, dst2, dst3,
         rows0, rows1, src_t, dst_t, rows_t, ones_v, ones_t, z_v,
         gsem0, gsem1, isS0, isS1, isD0, isD1, isD2, isD3,
         ss0, ss1, ss2, ss3, tsem_i, tsem_g) = rest
    else:
        (p_out, agg_sh, src0, src1, dst0, dst1, dst2, dst3,
         rows0, rows1, src_t, dst_t, rows_t,
         gsem0, gsem1, isS0, isS1, isD0, isD1, isD2, isD3,
         ss0, ss1, ss2, ss3, tsem_i, tsem_g) = rest
        c_out = cnt_sh = ones_v = ones_t = z_v = None
    srcs = (src0, src1)
    dsts = (dst0, dst1, dst2, dst3)
    rows = (rows0, rows1)
    gsems = (gsem0, gsem1)
    isems_s = (isS0, isS1)
    isems_d = (isD0, isD1, isD2, isD3)
    ssems = (ss0, ss1, ss2, ss3)

    cid = lax.axis_index("c")
    sid = lax.axis_index("s")
    wid = sid * NC + cid
    ebase = wid * PER_TILE
    tbase = ebase + N_FULL * CHUNK

    def start_src(g, p):
        pltpu.async_copy(src_hbm.at[pl.ds(ebase + g * CHUNK, CHUNK)],
                         srcs[p], isems_s[p])

    def start_dst(g, q):
        pltpu.async_copy(dst_hbm.at[pl.ds(ebase + g * CHUNK, CHUNK)],
                         dsts[q], isems_d[q])

    def wait_src(g, p):
        pltpu.make_async_copy(src_hbm.at[pl.ds(ebase + g * CHUNK, CHUNK)],
                              srcs[p], isems_s[p]).wait()

    def wait_dst(g, q):
        pltpu.make_async_copy(dst_hbm.at[pl.ds(ebase + g * CHUNK, CHUNK)],
                              dsts[q], isems_d[q]).wait()

    def start_gather(p):
        pltpu.async_copy(x_hbm.at[srcs[p]], rows[p], gsems[p])

    def wait_gather(p):
        pltpu.make_async_copy(x_hbm.at[srcs[p]], rows[p], gsems[p]).wait()

    def start_scat(q):
        pltpu.async_copy(rows[q % 2], agg_sh.at[dsts[q]], ssems[q], add=True)
        if with_cnt:
            pltpu.async_copy(ones_v, cnt_sh.at[dsts[q]], ssems[q], add=True)

    def wait_scat(q):
        pltpu.make_async_copy(rows[q % 2], agg_sh.at[dsts[q]], ssems[q]).wait()
        if with_cnt:
            pltpu.make_async_copy(ones_v, cnt_sh.at[dsts[q]], ssems[q]).wait()

    pltpu.async_copy(src_hbm.at[pl.ds(tbase, TAIL)], src_t, tsem_i)
    pltpu.async_copy(dst_hbm.at[pl.ds(tbase, TAIL)], dst_t, tsem_i)
    start_src(0, 0)
    start_dst(0, 0)
    start_src(1, 1)
    start_dst(1, 1)

    def zrows_body(r, carry):
        for j in range(D // 16):
            rows0[r, pl.ds(j * 16, 16)] = jnp.zeros((16,), jnp.float32)
        return carry

    lax.fori_loop(0, CHUNK, zrows_body, 0)

    if with_cnt:
        def zcnt_body(k, carry):
            z_v[pl.ds(k * 16, 16)] = jnp.zeros((16,), jnp.float32)
            return carry

        lax.fori_loop(0, ROWS_PER_TILE // 16, zcnt_body, 0)

        def ones_body(k, carry):
            ones_v[pl.ds(k * 16, 16)] = jnp.ones((16,), jnp.float32)
            return carry

        lax.fori_loop(0, CHUNK // 16, ones_body, 0)
        ones_t[pl.ds(0, 16)] = jnp.ones((16,), jnp.float32)

    row0 = sid * ROWS_PER_TILE
    for i in range(ROWS_PER_TILE // CHUNK):
        pltpu.sync_copy(rows0, agg_sh.at[pl.ds(row0 + i * CHUNK, CHUNK)])
    if with_cnt:
        pltpu.sync_copy(z_v, cnt_sh.at[pl.ds(row0, ROWS_PER_TILE)])

    plsc.subcore_barrier()

    def ops(g, k, first=False, n_left=3):
        p, q = k % 2, k
        if not first:
            wait_scat((q + 3) % 4)
        if n_left >= 1:
            wait_src(g + 1, (p + 1) % 2)
            wait_dst(g + 1, (q + 1) % 4)
            start_gather((p + 1) % 2)
        wait_gather(p)
        if n_left >= 2:
            start_src(g + 2, p)
        start_scat(q)
        if n_left >= 2:
            start_dst(g + 2, (q + 2) % 4)

    wait_src(0, 0)
    wait_dst(0, 0)
    start_gather(0)
    ops(0, 0, first=True)

    ITERS = (N_FULL - 5) // 4

    def body(i, carry):
        g0 = 4 * i + 1
        for k in range(4):
            ops(g0 + k, (1 + k) % 4)
        return carry

    lax.fori_loop(0, ITERS, body, 0)
    for g in range(4 * ITERS + 1, N_FULL):
        ops(g, g % 4, n_left=min(N_FULL - 1 - g, 3))
    wait_scat((N_FULL - 1) % 4)

    pltpu.make_async_copy(src_hbm.at[pl.ds(tbase, TAIL)], src_t, tsem_i).wait()
    pltpu.make_async_copy(dst_hbm.at[pl.ds(tbase, TAIL)], dst_t, tsem_i).wait()
    pltpu.async_copy(x_hbm.at[src_t], rows_t, tsem_g).wait()
    pltpu.sync_copy(rows_t, agg_sh.at[dst_t], add=True)
    if with_cnt:
        pltpu.sync_copy(ones_t, cnt_sh.at[dst_t], add=True)

    plsc.subcore_barrier()
    pltpu.sync_copy(agg_sh.at[pl.ds(row0, ROWS_PER_TILE)],
                    p_out.at[cid, pl.ds(row0, ROWS_PER_TILE)])
    if with_cnt:
        pltpu.sync_copy(cnt_sh.at[pl.ds(row0, ROWS_PER_TILE)],
                        c_out.at[cid, 0, pl.ds(row0, ROWS_PER_TILE)])


def _make_sc_agg(with_cnt):
    out_type = [jax.ShapeDtypeStruct((NC, NPAD, D), jnp.float32)]
    if with_cnt:
        out_type.append(jax.ShapeDtypeStruct((NC, 8, NPAD), jnp.float32))
    scratch = [
        pltpu.VMEM_SHARED((NPAD, D), jnp.float32),
    ]
    if with_cnt:
        scratch.append(pltpu.VMEM_SHARED((NPAD,), jnp.float32))
    scratch += [
        pltpu.VMEM((CHUNK,), jnp.int32),
        pltpu.VMEM((CHUNK,), jnp.int32),
        pltpu.VMEM((CHUNK,), jnp.int32),
        pltpu.VMEM((CHUNK,), jnp.int32),
        pltpu.VMEM((CHUNK,), jnp.int32),
        pltpu.VMEM((CHUNK,), jnp.int32),
        pltpu.VMEM((CHUNK, D), jnp.float32),
        pltpu.VMEM((CHUNK, D), jnp.float32),
        pltpu.VMEM((TAIL,), jnp.int32),
        pltpu.VMEM((TAIL,), jnp.int32),
        pltpu.VMEM((TAIL, D), jnp.float32),
    ]
    if with_cnt:
        scratch.append(pltpu.VMEM((CHUNK,), jnp.float32))
        scratch.append(pltpu.VMEM((TAIL,), jnp.float32))
        scratch.append(pltpu.VMEM((ROWS_PER_TILE,), jnp.float32))
    for _ in range(14):
        scratch.append(pltpu.SemaphoreType.DMA)

    return pl.kernel(
        functools.partial(_sc_agg_body, with_cnt),
        mesh=plsc.VectorSubcoreMesh(core_axis_name="c", subcore_axis_name="s"),
        out_type=out_type,
        scratch_types=scratch,
    )


_SC_AGG_CACHE = {}


def _get_sc_agg(with_cnt):
    if with_cnt not in _SC_AGG_CACHE:
        _SC_AGG_CACHE[with_cnt] = _make_sc_agg(with_cnt)
    return _SC_AGG_CACHE[with_cnt]


TC_BLOCK = 1024
NBLK = -(-N_NODES // TC_BLOCK)


def _tc_layer_body(relu, p_ref, c_ref, x_ref, wl_ref, wr_ref, b_ref, o_ref):
    agg = p_ref[0] + p_ref[1]
    cnt = (c_ref[0, 0, :] + c_ref[1, 0, :])[:, None]
    inv = 1.0 / jnp.maximum(cnt, 1.0)
    dn = (((1,), (1,)), ((), ()))
    acc = lax.dot_general(agg * inv, wl_ref[...], dn,
                          preferred_element_type=jnp.float32)
    acc += lax.dot_general(x_ref[...], wr_ref[...], dn,
                           preferred_element_type=jnp.float32)
    acc += b_ref[...]
    if relu:
        acc = jnp.maximum(acc, 0.0)
    o_ref[...] = acc


def _tc_layer(p, c3, x, wl, wr, b, relu):
    return pl.pallas_call(
        functools.partial(_tc_layer_body, relu),
        grid=(NBLK,),
        in_specs=[
            pl.BlockSpec((NC, TC_BLOCK, D), lambda i: (0, i, 0)),
            pl.BlockSpec((NC, 8, TC_BLOCK), lambda i: (0, 0, i)),
            pl.BlockSpec((TC_BLOCK, D), lambda i: (i, 0)),
            pl.BlockSpec((D, D), lambda i: (0, 0)),
            pl.BlockSpec((D, D), lambda i: (0, 0)),
            pl.BlockSpec((1, D), lambda i: (0, 0)),
        ],
        out_specs=pl.BlockSpec((TC_BLOCK, D), lambda i: (i, 0)),
        out_shape=jax.ShapeDtypeStruct((N_NODES, D), jnp.float32),
    )(p, c3, x, wl, wr, b.reshape(1, D))


def kernel(x, edge_index, Wl1, Wr1, b1, Wl2, Wr2, b2):
    src = edge_index[0].astype(jnp.int32)
    dst = edge_index[1].astype(jnp.int32)
    p1, craw = _get_sc_agg(True)(x, src, dst)
    h = _tc_layer(p1, craw, x, Wl1, Wr1, b1, relu=True)
    (p2,) = _get_sc_agg(False)(h, src, dst)
    out = _tc_layer(p2, craw, h, Wl2, Wr2, b2, relu=False)
    return out


# speedup vs baseline: 14.7826x; 1.0208x over previous
import functools

import jax
import jax.numpy as jnp
from jax import lax
from jax.experimental import pallas as pl
from jax.experimental.pallas import tpu as pltpu
from jax.experimental.pallas import tpu_sc as plsc

N_NODES = 10000
N_EDGES = 320000
D = 128

NC = 2
NS = 16
NW = NC * NS
PER_TILE = N_EDGES // NW
CHUNK = 128
N_FULL = PER_TILE // CHUNK
TAIL = PER_TILE - N_FULL * CHUNK
NPAD = 10240
ROWS_PER_TILE = NPAD // NS


def _sc_agg_body(with_cnt, x_hbm, src_hbm, dst_hbm, *rest):
    if with_cnt:
        (p_out, c_out, agg_sh, cnt_sh, src0, src1, dst0, dst1, dst2, dst3,
         rows0, rows1, src_t, dst_t, rows_t, ones_v, ones_t, z_v,
         gsem0, gsem1, isS0, isS1, isD0, isD1, isD2, isD3,
         ss0, ss1, ss2, ss3, tsem_i, tsem_g) = rest
    else:
        (p_out, agg_sh, src0, src1, dst0, dst1, dst2, dst3,
         rows0, rows1, src_t, dst_t, rows_t,
         gsem0, gsem1, isS0, isS1, isD0, isD1, isD2, isD3,
         ss0, ss1, ss2, ss3, tsem_i, tsem_g) = rest
        c_out = cnt_sh = ones_v = ones_t = z_v = None
    srcs = (src0, src1)
    dsts = (dst0, dst1, dst2, dst3)
    rows = (rows0, rows1)
    gsems = (gsem0, gsem1)
    isems_s = (isS0, isS1)
    isems_d = (isD0, isD1, isD2, isD3)
    ssems = (ss0, ss1, ss2, ss3)

    cid = lax.axis_index("c")
    sid = lax.axis_index("s")
    wid = sid * NC + cid
    ebase = wid * PER_TILE
    tbase = ebase + N_FULL * CHUNK

    def start_src(g, p):
        pltpu.async_copy(src_hbm.at[pl.ds(ebase + g * CHUNK, CHUNK)],
                         srcs[p], isems_s[p])

    def start_dst(g, q):
        pltpu.async_copy(dst_hbm.at[pl.ds(ebase + g * CHUNK, CHUNK)],
                         dsts[q], isems_d[q])

    def wait_src(g, p):
        pltpu.make_async_copy(src_hbm.at[pl.ds(ebase + g * CHUNK, CHUNK)],
                              srcs[p], isems_s[p]).wait()

    def wait_dst(g, q):
        pltpu.make_async_copy(dst_hbm.at[pl.ds(ebase + g * CHUNK, CHUNK)],
                              dsts[q], isems_d[q]).wait()

    def start_gather(p):
        pltpu.async_copy(x_hbm.at[srcs[p]], rows[p], gsems[p])

    def wait_gather(p):
        pltpu.make_async_copy(x_hbm.at[srcs[p]], rows[p], gsems[p]).wait()

    def start_scat(q):
        pltpu.async_copy(rows[q % 2], agg_sh.at[dsts[q]], ssems[q], add=True)
        if with_cnt:
            pltpu.async_copy(ones_v, cnt_sh.at[dsts[q]], ssems[q], add=True)

    def wait_scat(q):
        pltpu.make_async_copy(rows[q % 2], agg_sh.at[dsts[q]], ssems[q]).wait()
        if with_cnt:
            pltpu.make_async_copy(ones_v, cnt_sh.at[dsts[q]], ssems[q]).wait()

    pltpu.async_copy(src_hbm.at[pl.ds(tbase, TAIL)], src_t, tsem_i)
    pltpu.async_copy(dst_hbm.at[pl.ds(tbase, TAIL)], dst_t, tsem_i)
    start_src(0, 0)
    start_dst(0, 0)
    start_src(1, 1)
    start_dst(1, 1)

    def zrows_body(r, carry):
        for j in range(D // 16):
            rows1[r, pl.ds(j * 16, 16)] = jnp.zeros((16,), jnp.float32)
        return carry

    lax.fori_loop(0, CHUNK, zrows_body, 0)

    if with_cnt:
        def zcnt_body(k, carry):
            z_v[pl.ds(k * 16, 16)] = jnp.zeros((16,), jnp.float32)
            return carry

        lax.fori_loop(0, ROWS_PER_TILE // 16, zcnt_body, 0)

        def ones_body(k, carry):
            ones_v[pl.ds(k * 16, 16)] = jnp.ones((16,), jnp.float32)
            return carry

        lax.fori_loop(0, CHUNK // 16, ones_body, 0)
        ones_t[pl.ds(0, 16)] = jnp.ones((16,), jnp.float32)

    row0 = sid * ROWS_PER_TILE
    zcopies = []
    for i in range(ROWS_PER_TILE // CHUNK):
        zcopies.append(pltpu.async_copy(
            rows1, agg_sh.at[pl.ds(row0 + i * CHUNK, CHUNK)], tsem_g))
    if with_cnt:
        zcopies.append(pltpu.async_copy(
            z_v, cnt_sh.at[pl.ds(row0, ROWS_PER_TILE)], tsem_g))

    wait_src(0, 0)
    wait_dst(0, 0)
    start_gather(0)
    pltpu.make_async_copy(src_hbm.at[pl.ds(tbase, TAIL)], src_t, tsem_i).wait()
    pltpu.make_async_copy(dst_hbm.at[pl.ds(tbase, TAIL)], dst_t, tsem_i).wait()
    tail_gather = pltpu.async_copy(x_hbm.at[src_t], rows_t, tsem_i)

    for c in zcopies:
        c.wait()

    plsc.subcore_barrier()

    def ops(g, k, first=False, n_left=3):
        p, q = k % 2, k
        if not first:
            wait_scat((q + 3) % 4)
        if n_left >= 1:
            wait_src(g + 1, (p + 1) % 2)
            wait_dst(g + 1, (q + 1) % 4)
            start_gather((p + 1) % 2)
        wait_gather(p)
        if n_left >= 2:
            start_src(g + 2, p)
        start_scat(q)
        if n_left >= 2:
            start_dst(g + 2, (q + 2) % 4)

    ops(0, 0, first=True)

    ITERS = (N_FULL - 5) // 4

    def body(i, carry):
        g0 = 4 * i + 1
        for k in range(4):
            ops(g0 + k, (1 + k) % 4)
        return carry

    lax.fori_loop(0, ITERS, body, 0)
    for g in range(4 * ITERS + 1, N_FULL):
        ops(g, g % 4, n_left=min(N_FULL - 1 - g, 3))
    wait_scat((N_FULL - 1) % 4)

    tail_gather.wait()
    pltpu.sync_copy(rows_t, agg_sh.at[dst_t], add=True)
    if with_cnt:
        pltpu.sync_copy(ones_t, cnt_sh.at[dst_t], add=True)

    plsc.subcore_barrier()
    pltpu.sync_copy(agg_sh.at[pl.ds(row0, ROWS_PER_TILE)],
                    p_out.at[cid, pl.ds(row0, ROWS_PER_TILE)])
    if with_cnt:
        pltpu.sync_copy(cnt_sh.at[pl.ds(row0, ROWS_PER_TILE)],
                        c_out.at[cid, 0, pl.ds(row0, ROWS_PER_TILE)])


def _make_sc_agg(with_cnt):
    out_type = [jax.ShapeDtypeStruct((NC, NPAD, D), jnp.float32)]
    if with_cnt:
        out_type.append(jax.ShapeDtypeStruct((NC, 8, NPAD), jnp.float32))
    scratch = [
        pltpu.VMEM_SHARED((NPAD, D), jnp.float32),
    ]
    if with_cnt:
        scratch.append(pltpu.VMEM_SHARED((NPAD,), jnp.float32))
    scratch += [
        pltpu.VMEM((CHUNK,), jnp.int32),
        pltpu.VMEM((CHUNK,), jnp.int32),
        pltpu.VMEM((CHUNK,), jnp.int32),
        pltpu.VMEM((CHUNK,), jnp.int32),
        pltpu.VMEM((CHUNK,), jnp.int32),
        pltpu.VMEM((CHUNK,), jnp.int32),
        pltpu.VMEM((CHUNK, D), jnp.float32),
        pltpu.VMEM((CHUNK, D), jnp.float32),
        pltpu.VMEM((TAIL,), jnp.int32),
        pltpu.VMEM((TAIL,), jnp.int32),
        pltpu.VMEM((TAIL, D), jnp.float32),
    ]
    if with_cnt:
        scratch.append(pltpu.VMEM((CHUNK,), jnp.float32))
        scratch.append(pltpu.VMEM((TAIL,), jnp.float32))
        scratch.append(pltpu.VMEM((ROWS_PER_TILE,), jnp.float32))
    for _ in range(14):
        scratch.append(pltpu.SemaphoreType.DMA)

    return pl.kernel(
        functools.partial(_sc_agg_body, with_cnt),
        mesh=plsc.VectorSubcoreMesh(core_axis_name="c", subcore_axis_name="s"),
        out_type=out_type,
        scratch_types=scratch,
    )


_SC_AGG_CACHE = {}


def _get_sc_agg(with_cnt):
    if with_cnt not in _SC_AGG_CACHE:
        _SC_AGG_CACHE[with_cnt] = _make_sc_agg(with_cnt)
    return _SC_AGG_CACHE[with_cnt]


TC_BLOCK = 1024
NBLK = -(-N_NODES // TC_BLOCK)


def _tc_layer_body(relu, p_ref, c_ref, x_ref, wl_ref, wr_ref, b_ref, o_ref):
    agg = p_ref[0] + p_ref[1]
    cnt = (c_ref[0, 0, :] + c_ref[1, 0, :])[:, None]
    inv = 1.0 / jnp.maximum(cnt, 1.0)
    dn = (((1,), (1,)), ((), ()))
    acc = lax.dot_general(agg * inv, wl_ref[...], dn,
                          preferred_element_type=jnp.float32)
    acc += lax.dot_general(x_ref[...], wr_ref[...], dn,
                           preferred_element_type=jnp.float32)
    acc += b_ref[...]
    if relu:
        acc = jnp.maximum(acc, 0.0)
    o_ref[...] = acc


def _tc_layer(p, c3, x, wl, wr, b, relu):
    return pl.pallas_call(
        functools.partial(_tc_layer_body, relu),
        grid=(NBLK,),
        in_specs=[
            pl.BlockSpec((NC, TC_BLOCK, D), lambda i: (0, i, 0)),
            pl.BlockSpec((NC, 8, TC_BLOCK), lambda i: (0, 0, i)),
            pl.BlockSpec((TC_BLOCK, D), lambda i: (i, 0)),
            pl.BlockSpec((D, D), lambda i: (0, 0)),
            pl.BlockSpec((D, D), lambda i: (0, 0)),
            pl.BlockSpec((1, D), lambda i: (0, 0)),
        ],
        out_specs=pl.BlockSpec((TC_BLOCK, D), lambda i: (i, 0)),
        out_shape=jax.ShapeDtypeStruct((N_NODES, D), jnp.float32),
    )(p, c3, x, wl, wr, b.reshape(1, D))


def kernel(x, edge_index, Wl1, Wr1, b1, Wl2, Wr2, b2):
    src = edge_index[0].astype(jnp.int32)
    dst = edge_index[1].astype(jnp.int32)
    p1, craw = _get_sc_agg(True)(x, src, dst)
    h = _tc_layer(p1, craw, x, Wl1, Wr1, b1, relu=True)
    (p2,) = _get_sc_agg(False)(h, src, dst)
    out = _tc_layer(p2, craw, h, Wl2, Wr2, b2, relu=False)
    return out
